# Initial kernel scaffold; baseline (speedup 1.0000x reference)
import jax
import jax.numpy as jnp
from jax.experimental import pallas as pl


def kernel(loc_data, conf_data, priors, targets):
    raise NotImplementedError("write your pallas kernel here")


# speedup vs baseline: 12.0715x; 12.0715x over previous
import functools

import jax
import jax.numpy as jnp
from jax.experimental import pallas as pl
from jax.experimental.pallas import tpu as pltpu

_NUM_CLASSES = 21
_THRESHOLD = 0.5
_NEGPOS_RATIO = 3
_VAR0 = 0.1
_VAR1 = 0.2


def _smooth_l1(d):
    ad = jnp.abs(d)
    return jnp.where(ad < 1.0, 0.5 * d * d, ad - 0.5)


def _mbox_kernel(loc_ref, conf_ref, priors_ref, targets_ref,
                 ll_ref, lc_ref, n_ref):
    b = pl.program_id(0)
    num_priors = loc_ref.shape[1]
    num_objs = targets_ref.shape[0]

    @pl.when(b == 0)
    def _init():
        ll_ref[0, 0] = 0.0
        lc_ref[0, 0] = 0.0
        n_ref[0, 0] = 0.0

    px = priors_ref[0:1, :]
    py = priors_ref[1:2, :]
    pw = priors_ref[2:3, :]
    ph = priors_ref[3:4, :]
    p_x1 = px - pw * 0.5
    p_y1 = py - ph * 0.5
    p_x2 = px + pw * 0.5
    p_y2 = py + ph * 0.5
    area_b = (p_x2 - p_x1) * (p_y2 - p_y1)

    t_x1 = targets_ref[:, 0:1]
    t_y1 = targets_ref[:, 1:2]
    t_x2 = targets_ref[:, 2:3]
    t_y2 = targets_ref[:, 3:4]
    area_a = (t_x2 - t_x1) * (t_y2 - t_y1)

    ix = jnp.clip(jnp.minimum(t_x2, p_x2) - jnp.maximum(t_x1, p_x1), 0.0, None)
    iy = jnp.clip(jnp.minimum(t_y2, p_y2) - jnp.maximum(t_y1, p_y1), 0.0, None)
    inter = ix * iy
    overlaps = inter / (area_a + area_b - inter)

    lane_iota = jax.lax.broadcasted_iota(jnp.int32, (1, num_priors), 1)

    bto = jnp.max(overlaps, axis=0, keepdims=True)
    bti = jnp.zeros((1, num_priors), jnp.int32)
    for t in range(num_objs - 1, 0, -1):
        bti = jnp.where(overlaps[t:t + 1, :] == bto, t, bti)

    row_max = jnp.max(overlaps, axis=1, keepdims=True)
    big = jnp.int32(num_priors)
    bpi = jnp.min(jnp.where(overlaps == row_max, lane_iota, big),
                  axis=1, keepdims=True)

    for t in range(num_objs):
        m = lane_iota == bpi[t, 0]
        bto = jnp.where(m, 2.0, bto)
        bti = jnp.where(m, t, bti)

    m_x1 = jnp.full((1, num_priors), targets_ref[0, 0])
    m_y1 = jnp.full((1, num_priors), targets_ref[0, 1])
    m_x2 = jnp.full((1, num_priors), targets_ref[0, 2])
    m_y2 = jnp.full((1, num_priors), targets_ref[0, 3])
    m_lb = jnp.full((1, num_priors), targets_ref[0, 4])
    for t in range(1, num_objs):
        sel = bti == t
        m_x1 = jnp.where(sel, targets_ref[t, 0], m_x1)
        m_y1 = jnp.where(sel, targets_ref[t, 1], m_y1)
        m_x2 = jnp.where(sel, targets_ref[t, 2], m_x2)
        m_y2 = jnp.where(sel, targets_ref[t, 3], m_y2)
        m_lb = jnp.where(sel, targets_ref[t, 4], m_lb)

    pos = bto >= _THRESHOLD
    posf = pos.astype(jnp.float32)
    num_pos = jnp.sum(pos.astype(jnp.int32))
    tgt_class = jnp.where(pos, m_lb.astype(jnp.int32) + 1, 0)

    g_cx = ((m_x1 + m_x2) * 0.5 - px) / (_VAR0 * pw)
    g_cy = ((m_y1 + m_y2) * 0.5 - py) / (_VAR0 * ph)
    g_w = jnp.log((m_x2 - m_x1) / pw) * (1.0 / _VAR1)
    g_h = jnp.log((m_y2 - m_y1) / ph) * (1.0 / _VAR1)
    ll = (_smooth_l1(loc_ref[0:1, :] - g_cx) +
          _smooth_l1(loc_ref[1:2, :] - g_cy) +
          _smooth_l1(loc_ref[2:3, :] - g_w) +
          _smooth_l1(loc_ref[3:4, :] - g_h))
    loss_l_row = jnp.sum(ll * posf)

    conf = conf_ref[:, :]
    cmax = jnp.max(conf, axis=0, keepdims=True)
    ssum = jnp.sum(jnp.exp(conf - cmax), axis=0, keepdims=True)
    class_iota = jax.lax.broadcasted_iota(
        jnp.int32, (conf.shape[0], num_priors), 0)
    conf_tgt = jnp.sum(jnp.where(class_iota == tgt_class, conf, 0.0),
                       axis=0, keepdims=True)
    ce = jnp.log(ssum) + cmax - conf_tgt

    loss_c_pos = jnp.sum(ce * posf)

    mine = jnp.maximum(jnp.where(pos, 0.0, ce), 0.0)
    v_int = jax.lax.bitcast_convert_type(mine, jnp.int32)
    k = jnp.minimum(_NEGPOS_RATIO * num_pos, num_priors - 1)

    def bs_body(_, carry):
        lo, hi = carry
        mid = lo + (hi - lo) // 2
        cnt = jnp.sum((v_int >= mid).astype(jnp.int32))
        ge = cnt >= k
        return jnp.where(ge, mid, lo), jnp.where(ge, hi, mid)

    lo, hi = jax.lax.fori_loop(
        0, 31, bs_body, (jnp.int32(0), jnp.int32(2139095041)))
    t_val = jax.lax.bitcast_convert_type(lo, jnp.float32)
    gt = v_int > lo
    cnt_gt = jnp.sum(gt.astype(jnp.int32))
    sum_gt = jnp.sum(jnp.where(gt, mine, 0.0))
    topk = sum_gt + (k - cnt_gt).astype(jnp.float32) * t_val

    ll_ref[0, 0] += loss_l_row
    lc_ref[0, 0] += loss_c_pos + topk
    n_ref[0, 0] += num_pos.astype(jnp.float32)


@jax.jit
def kernel(loc_data, conf_data, priors, targets):
    num, num_priors, _ = loc_data.shape
    num_classes = conf_data.shape[2]
    num_objs = targets.shape[1]

    loc_t = loc_data.transpose(0, 2, 1)
    conf_t = conf_data.transpose(0, 2, 1)
    priors_t = priors.T

    out_shape = [jax.ShapeDtypeStruct((1, 1), jnp.float32)] * 3
    ll, lc, n = pl.pallas_call(
        _mbox_kernel,
        grid=(num,),
        in_specs=[
            pl.BlockSpec((None, 4, num_priors), lambda b: (b, 0, 0)),
            pl.BlockSpec((None, num_classes, num_priors), lambda b: (b, 0, 0)),
            pl.BlockSpec((4, num_priors), lambda b: (0, 0)),
            pl.BlockSpec((None, num_objs, 5), lambda b: (b, 0, 0)),
        ],
        out_specs=[
            pl.BlockSpec(memory_space=pltpu.SMEM),
            pl.BlockSpec(memory_space=pltpu.SMEM),
            pl.BlockSpec(memory_space=pltpu.SMEM),
        ],
        out_shape=out_shape,
    )(loc_t, conf_t, priors_t, targets)

    n = n[0, 0]
    return (ll[0, 0] / n, lc[0, 0] / n)


# speedup vs baseline: 19.5675x; 1.6210x over previous
import jax
import jax.numpy as jnp
from jax.experimental import pallas as pl
from jax.experimental.pallas import tpu as pltpu

_THRESHOLD = 0.5
_NEGPOS_RATIO = 3
_VAR0 = 0.1
_VAR1 = 0.2
_ROWS = 8


def _smooth_l1(d):
    ad = jnp.abs(d)
    return jnp.where(ad < 1.0, 0.5 * d * d, ad - 0.5)


def _mbox_kernel(loc_ref, conf_ref, priors_ref, targets_ref,
                 ll_ref, lc_ref, n_ref):
    b = pl.program_id(0)
    R = _ROWS
    C = conf_ref.shape[0]
    P = loc_ref.shape[2]
    T = targets_ref.shape[1]

    @pl.when(b == 0)
    def _init():
        ll_ref[0, 0] = 0.0
        lc_ref[0, 0] = 0.0
        n_ref[0, 0] = 0.0

    px = priors_ref[0:1, :]
    py = priors_ref[1:2, :]
    pw = priors_ref[2:3, :]
    ph = priors_ref[3:4, :]
    p_x1 = px - pw * 0.5
    p_y1 = py - ph * 0.5
    p_x2 = px + pw * 0.5
    p_y2 = py + ph * 0.5
    area_b = (p_x2 - p_x1) * (p_y2 - p_y1)

    t_x1 = targets_ref[:, :, 0:1]
    t_y1 = targets_ref[:, :, 1:2]
    t_x2 = targets_ref[:, :, 2:3]
    t_y2 = targets_ref[:, :, 3:4]
    area_a = (t_x2 - t_x1) * (t_y2 - t_y1)

    p3 = lambda a: a[None]
    ix = jnp.clip(jnp.minimum(t_x2, p3(p_x2)) - jnp.maximum(t_x1, p3(p_x1)),
                  0.0, None)
    iy = jnp.clip(jnp.minimum(t_y2, p3(p_y2)) - jnp.maximum(t_y1, p3(p_y1)),
                  0.0, None)
    inter = ix * iy
    overlaps = inter / (area_a + p3(area_b) - inter)

    bto = jnp.max(overlaps, axis=1)
    bti = jnp.zeros((R, P), jnp.int32)
    for t in range(T - 1, 0, -1):
        bti = jnp.where(overlaps[:, t, :] == bto, t, bti)

    lane3 = jax.lax.broadcasted_iota(jnp.int32, (R, T, P), 2)
    row_max = jnp.max(overlaps, axis=2, keepdims=True)
    bpi = jnp.min(jnp.where(overlaps == row_max, lane3, P),
                  axis=2, keepdims=True)

    mask3 = lane3 == bpi
    tio3 = jax.lax.broadcasted_iota(jnp.int32, (R, T, P), 1)
    forced = jnp.max(jnp.where(mask3, tio3, -1), axis=1)
    has_f = forced >= 0
    bto = jnp.where(has_f, 2.0, bto)
    bti = jnp.where(has_f, forced, bti)

    pos = bto >= _THRESHOLD
    posf = pos.astype(jnp.float32)
    num_pos = jnp.sum(posf, axis=1, keepdims=True)

    m_x1 = jnp.broadcast_to(t_x1[:, 0, :], (R, P))
    m_y1 = jnp.broadcast_to(t_y1[:, 0, :], (R, P))
    m_x2 = jnp.broadcast_to(t_x2[:, 0, :], (R, P))
    m_y2 = jnp.broadcast_to(t_y2[:, 0, :], (R, P))
    m_lb = jnp.broadcast_to(targets_ref[:, 0, 4:5], (R, P))
    for t in range(1, T):
        sel = bti == t
        m_x1 = jnp.where(sel, t_x1[:, t, :], m_x1)
        m_y1 = jnp.where(sel, t_y1[:, t, :], m_y1)
        m_x2 = jnp.where(sel, t_x2[:, t, :], m_x2)
        m_y2 = jnp.where(sel, t_y2[:, t, :], m_y2)
        m_lb = jnp.where(sel, targets_ref[:, t, 4:5], m_lb)

    tgt_class = jnp.where(pos, m_lb.astype(jnp.int32) + 1, 0)

    g_cx = ((m_x1 + m_x2) * 0.5 - px) / (_VAR0 * pw)
    g_cy = ((m_y1 + m_y2) * 0.5 - py) / (_VAR0 * ph)
    g_w = jnp.log((m_x2 - m_x1) / pw) * (1.0 / _VAR1)
    g_h = jnp.log((m_y2 - m_y1) / ph) * (1.0 / _VAR1)
    ll = (_smooth_l1(loc_ref[0] - g_cx) + _smooth_l1(loc_ref[1] - g_cy) +
          _smooth_l1(loc_ref[2] - g_w) + _smooth_l1(loc_ref[3] - g_h))
    loss_l_step = jnp.sum(ll * posf)

    cmax = conf_ref[0]
    for c in range(1, C):
        cmax = jnp.maximum(cmax, conf_ref[c])
    ssum = jnp.exp(conf_ref[0] - cmax)
    conf_tgt = conf_ref[0]
    for c in range(1, C):
        x = conf_ref[c]
        ssum = ssum + jnp.exp(x - cmax)
        conf_tgt = jnp.where(tgt_class == c, x, conf_tgt)
    ce = jnp.log(ssum) + cmax - conf_tgt

    loss_c_pos = jnp.sum(ce * posf)

    mine = jnp.maximum(jnp.where(pos, 0.0, ce), 0.0)
    v_int = jax.lax.bitcast_convert_type(mine, jnp.int32)
    k = jnp.minimum(
        _NEGPOS_RATIO * jnp.sum(pos.astype(jnp.int32), axis=1, keepdims=True),
        P - 1)

    def bs_body(_, carry):
        lo, hi = carry
        mid = lo + (hi - lo) // 2
        cnt = jnp.sum((v_int >= mid).astype(jnp.int32), axis=1, keepdims=True)
        ge = cnt >= k
        return jnp.where(ge, mid, lo), jnp.where(ge, hi, mid)

    lo0 = jnp.zeros((R, 1), jnp.int32)
    hi0 = jnp.full((R, 1), 2139095041, jnp.int32)
    lo, hi = jax.lax.fori_loop(0, 31, bs_body, (lo0, hi0))
    t_val = jax.lax.bitcast_convert_type(lo, jnp.float32)
    gt = v_int > lo
    cnt_gt = jnp.sum(gt.astype(jnp.int32), axis=1, keepdims=True)
    sum_gt = jnp.sum(jnp.where(gt, mine, 0.0), axis=1, keepdims=True)
    topk = sum_gt + (k - cnt_gt).astype(jnp.float32) * t_val

    ll_ref[0, 0] += loss_l_step
    lc_ref[0, 0] += loss_c_pos + jnp.sum(topk)
    n_ref[0, 0] += jnp.sum(num_pos)


@jax.jit
def kernel(loc_data, conf_data, priors, targets):
    num, num_priors, _ = loc_data.shape
    num_classes = conf_data.shape[2]
    num_objs = targets.shape[1]

    loc_t = loc_data.transpose(2, 0, 1)
    conf_t = conf_data.transpose(2, 0, 1)
    priors_t = priors.T

    out_shape = [jax.ShapeDtypeStruct((1, 1), jnp.float32)] * 3
    ll, lc, n = pl.pallas_call(
        _mbox_kernel,
        grid=(num // _ROWS,),
        in_specs=[
            pl.BlockSpec((4, _ROWS, num_priors), lambda b: (0, b, 0)),
            pl.BlockSpec((num_classes, _ROWS, num_priors), lambda b: (0, b, 0)),
            pl.BlockSpec((4, num_priors), lambda b: (0, 0)),
            pl.BlockSpec((_ROWS, num_objs, 5), lambda b: (b, 0, 0)),
        ],
        out_specs=[
            pl.BlockSpec(memory_space=pltpu.SMEM),
            pl.BlockSpec(memory_space=pltpu.SMEM),
            pl.BlockSpec(memory_space=pltpu.SMEM),
        ],
        out_shape=out_shape,
    )(loc_t, conf_t, priors_t, targets)

    n = n[0, 0]
    return (ll[0, 0] / n, lc[0, 0] / n)
